# baseline (speedup 1.0000x reference)
import functools

import jax
import jax.numpy as jnp
from jax import lax
from jax.experimental import pallas as pl
from jax.experimental.pallas import tpu as pltpu
from jax.experimental.pallas import tpu_sc as plsc

NPIX = 2048
NCELL = NPIX * NPIX
N = 4194304
SENT = NCELL

HALF = NCELL // 2
PACKED = HALF // 2
NDUMP = 2048
HSIZE = PACKED + NDUMP
INIT_PER_TILE = HSIZE // 16
OUT_PER_TILE = PACKED // 16

B_H = 4096
SCAN_PER_TILE = N // 16

B_G = 4096
G_PER_TILE = N // 32


def _idx_body(u_ref, v_ref, ind_ref, mask_ref):
    u = u_ref[...]
    v = v_ref[...]
    fl = v < 0.0
    us = jnp.where(fl, -u, u)
    vs = jnp.where(fl, -v, v)
    p = jnp.floor((us + 1.0) * (NPIX / 2.0)).astype(jnp.int32)
    q = jnp.floor((vs + 1.0) * (NPIX / 2.0)).astype(jnp.int32)
    valid = (p >= 0) & (p < NPIX) & (q >= 0) & (q < NPIX)
    ind = jnp.where(valid, p * NPIX + jnp.where(valid, q, 0), SENT)
    ind_ref[...] = ind
    mask_ref[...] = valid


_IDX_BLOCK = 524288

_idx_call = pl.pallas_call(
    _idx_body,
    grid=(N // _IDX_BLOCK,),
    in_specs=[pl.BlockSpec((_IDX_BLOCK,), lambda i: (i,)),
              pl.BlockSpec((_IDX_BLOCK,), lambda i: (i,))],
    out_specs=[pl.BlockSpec((_IDX_BLOCK,), lambda i: (i,)),
               pl.BlockSpec((_IDX_BLOCK,), lambda i: (i,))],
    out_shape=[jax.ShapeDtypeStruct((N,), jnp.int32),
               jax.ShapeDtypeStruct((N,), jnp.bool_)],
)


_NCH = SCAN_PER_TILE // B_H


def _hist_body(ind_hbm, zeros_hbm, out_hbm, hist_sh,
               stage0, stage1, cidx0, cidx1, cval0, cval1,
               isem0, isem1, fsem0, fsem1):
    c = lax.axis_index("c")
    s = lax.axis_index("s")
    base = c * HALF
    iota = lax.iota(jnp.int32, 16)
    bufs = [(stage0, cidx0, cval0, isem0, fsem0),
            (stage1, cidx1, cval1, isem1, fsem1)]

    pltpu.sync_copy(zeros_hbm.at[pl.ds(s * INIT_PER_TILE, INIT_PER_TILE)],
                    hist_sh.at[pl.ds(s * INIT_PER_TILE, INIT_PER_TILE)])
    plsc.subcore_barrier()

    dump_base = PACKED + s * 128

    def off(kk):
        return s * SCAN_PER_TILE + kk * B_H

    for b in range(2):
        stage, ci, cv, isem, fsem = bufs[b]
        pltpu.async_copy(ind_hbm.at[pl.ds(off(b), B_H)], stage, isem)

    def pair(k2, _):
        for b in range(2):
            stage, ci, cv, isem, fsem = bufs[b]
            kk = k2 * 2 + b
            pltpu.make_async_copy(
                ind_hbm.at[pl.ds(0, B_H)], stage, isem).wait()

            @pl.when(kk >= 2)
            def _credit():
                pltpu.make_async_copy(cv, hist_sh.at[ci], fsem).wait()

            def vec(j, _):
                iv = stage[pl.ds(j * 16, 16)]
                local = iv - base
                ok = (local >= 0) & (local < HALF)
                word = local & (PACKED - 1)
                dump = dump_base + ((j * 16 + iota) & 127)
                ci[pl.ds(j * 16, 16)] = jnp.where(ok, word, dump)
                cv[pl.ds(j * 16, 16)] = jnp.where(
                    ok, jnp.where(local >= PACKED, 4096.0, 1.0), 0.0)
                return 0
            lax.fori_loop(0, B_H // 16, vec, 0)

            pltpu.async_copy(cv, hist_sh.at[ci], fsem, add=True)

            @pl.when(kk + 2 < _NCH)
            def _prefetch():
                pltpu.async_copy(ind_hbm.at[pl.ds(off(kk + 2), B_H)],
                                 stage, isem)
        return 0
    lax.fori_loop(0, _NCH // 2, pair, 0)

    for b in range(2):
        stage, ci, cv, isem, fsem = bufs[b]
        pltpu.make_async_copy(cv, hist_sh.at[ci], fsem).wait()
    plsc.subcore_barrier()

    pltpu.sync_copy(hist_sh.at[pl.ds(s * OUT_PER_TILE, OUT_PER_TILE)],
                    out_hbm.at[pl.ds(c * PACKED + s * OUT_PER_TILE,
                                     OUT_PER_TILE)])


_hist_call = functools.partial(
    pl.kernel,
    out_type=jax.ShapeDtypeStruct((2 * PACKED,), jnp.float32),
    mesh=plsc.VectorSubcoreMesh(core_axis_name="c", subcore_axis_name="s"),
    scratch_types=[
        pltpu.VMEM_SHARED((HSIZE,), jnp.float32),
        pltpu.VMEM((B_H,), jnp.int32),
        pltpu.VMEM((B_H,), jnp.int32),
        pltpu.VMEM((B_H,), jnp.int32),
        pltpu.VMEM((B_H,), jnp.int32),
        pltpu.VMEM((B_H,), jnp.float32),
        pltpu.VMEM((B_H,), jnp.float32),
        pltpu.SemaphoreType.DMA,
        pltpu.SemaphoreType.DMA,
        pltpu.SemaphoreType.DMA,
        pltpu.SemaphoreType.DMA,
    ],
)(_hist_body)


_BAND = 256
_NB = NPIX // _BAND


def _pk(k):
    return (k & 1) + 2 * (k >> 2)


def _use_b(k):
    return ((k >> 1) & 1) == 1


def _unpack(xp, use_b):
    b = jnp.floor(xp * (1.0 / 4096.0))
    a = xp - b * 4096.0
    return jnp.where(use_b, b, a)


def _conv_body(prev_ref, cur_ref, next_ref, o_ref):
    k = pl.program_id(0)
    x = _unpack(cur_ref[...], _use_b(k))
    top2 = _unpack(prev_ref[_BAND - 2:, :], _use_b(k - 1))
    top2 = jnp.where(k == 0, jnp.zeros_like(top2), top2)
    bot2 = _unpack(next_ref[:2, :], _use_b(k + 1))
    bot2 = jnp.where(k == _NB - 1, jnp.zeros_like(bot2), bot2)
    ext = jnp.concatenate([top2, x, bot2], axis=0)
    z1 = jnp.zeros((_BAND + 4, 1), jnp.float32)
    z2 = jnp.zeros((_BAND + 4, 2), jnp.float32)
    rs = ext
    rs = rs + jnp.concatenate([ext[:, 1:], z1], axis=1)
    rs = rs + jnp.concatenate([z1, ext[:, :-1]], axis=1)
    rs = rs + jnp.concatenate([ext[:, 2:], z2], axis=1)
    rs = rs + jnp.concatenate([z2, ext[:, :-2]], axis=1)
    cs = (rs[0:_BAND] + rs[1:_BAND + 1] + rs[2:_BAND + 2]
          + rs[3:_BAND + 3] + rs[4:_BAND + 4])
    o_ref[...] = cs * (1.0 / 25.0)


_conv_call = pl.pallas_call(
    _conv_body,
    grid=(_NB,),
    in_specs=[
        pl.BlockSpec((_BAND, NPIX), lambda k: (_pk(jnp.maximum(k - 1, 0)), 0)),
        pl.BlockSpec((_BAND, NPIX), lambda k: (_pk(k), 0)),
        pl.BlockSpec((_BAND, NPIX),
                     lambda k: (_pk(jnp.minimum(k + 1, _NB - 1)), 0)),
    ],
    out_specs=pl.BlockSpec((_BAND, NPIX), lambda k: (k, 0)),
    out_shape=jax.ShapeDtypeStruct((NPIX, NPIX), jnp.float32),
)


_NCG = G_PER_TILE // B_G


def _gather_body(ind_hbm, sm_hbm, w_hbm,
                 stage0, stage1, lidx0, lidx1, dens0, dens1, wbuf0, wbuf1,
                 isem0, isem1, gsem0, gsem1, osem0, osem1):
    c = lax.axis_index("c")
    s = lax.axis_index("s")
    wid = c * 16 + s
    bufs = [(stage0, lidx0, dens0, wbuf0, isem0, gsem0, osem0),
            (stage1, lidx1, dens1, wbuf1, isem1, gsem1, osem1)]

    def off(kk):
        return wid * G_PER_TILE + kk * B_G

    def compute_w(stage, dens, wbuf):
        def vec2(j, _):
            dv = dens[pl.ds(j * 16, 16)]
            iv = stage[pl.ds(j * 16, 16)]
            w = jnp.where(iv < SENT, 1.0 / jnp.maximum(dv, 1e-8), 0.0)
            wbuf[pl.ds(j * 16, 16)] = w
            return 0
        lax.fori_loop(0, B_G // 16, vec2, 0)

    pltpu.async_copy(ind_hbm.at[pl.ds(off(0), B_G)], stage0, isem0)

    def pair(k2, _):
        for b in range(2):
            stage, lidx, dens, wbuf, isem, gsem, osem = bufs[b]
            stage_q, lidx_q, dens_q, wbuf_q, isem_q, gsem_q, osem_q = \
                bufs[1 - b]
            kk = k2 * 2 + b
            pltpu.make_async_copy(
                ind_hbm.at[pl.ds(0, B_G)], stage, isem).wait()

            def vec1(j, _):
                iv = stage[pl.ds(j * 16, 16)]
                lidx[pl.ds(j * 16, 16)] = jnp.minimum(iv, NCELL - 1)
                return 0
            lax.fori_loop(0, B_G // 16, vec1, 0)
            pltpu.async_copy(sm_hbm.at[lidx], dens, gsem)

            @pl.when(kk >= 1)
            def _finish_prev():
                pltpu.make_async_copy(
                    sm_hbm.at[lidx_q], dens_q, gsem_q).wait()

                @pl.when(kk >= 3)
                def _drain_out():
                    pltpu.make_async_copy(
                        wbuf_q, w_hbm.at[pl.ds(0, B_G)], osem_q).wait()
                compute_w(stage_q, dens_q, wbuf_q)
                pltpu.async_copy(wbuf_q, w_hbm.at[pl.ds(off(kk - 1), B_G)],
                                 osem_q)

            @pl.when(kk + 1 < _NCG)
            def _prefetch():
                pltpu.async_copy(ind_hbm.at[pl.ds(off(kk + 1), B_G)],
                                 stage_q, isem_q)
        return 0
    lax.fori_loop(0, _NCG // 2, pair, 0)

    stage, lidx, dens, wbuf, isem, gsem, osem = bufs[(_NCG - 1) % 2]
    pltpu.make_async_copy(sm_hbm.at[lidx], dens, gsem).wait()
    pltpu.make_async_copy(wbuf, w_hbm.at[pl.ds(0, B_G)], osem).wait()
    compute_w(stage, dens, wbuf)
    pltpu.async_copy(wbuf, w_hbm.at[pl.ds(off(_NCG - 1), B_G)], osem)
    for b in range(2):
        stage, lidx, dens, wbuf, isem, gsem, osem = bufs[b]
        pltpu.make_async_copy(wbuf, w_hbm.at[pl.ds(0, B_G)], osem).wait()


_gather_call = functools.partial(
    pl.kernel,
    out_type=jax.ShapeDtypeStruct((N,), jnp.float32),
    mesh=plsc.VectorSubcoreMesh(core_axis_name="c", subcore_axis_name="s"),
    scratch_types=[
        pltpu.VMEM((B_G,), jnp.int32),
        pltpu.VMEM((B_G,), jnp.int32),
        pltpu.VMEM((B_G,), jnp.int32),
        pltpu.VMEM((B_G,), jnp.int32),
        pltpu.VMEM((B_G,), jnp.float32),
        pltpu.VMEM((B_G,), jnp.float32),
        pltpu.VMEM((B_G,), jnp.float32),
        pltpu.VMEM((B_G,), jnp.float32),
        pltpu.SemaphoreType.DMA,
        pltpu.SemaphoreType.DMA,
        pltpu.SemaphoreType.DMA,
        pltpu.SemaphoreType.DMA,
        pltpu.SemaphoreType.DMA,
        pltpu.SemaphoreType.DMA,
    ],
)(_gather_body)


def kernel(u, v):
    ind, mask = _idx_call(u, v)
    zeros32 = jnp.zeros((HSIZE,), jnp.float32)
    hist = _hist_call(ind, zeros32)
    ph = hist.reshape(NPIX // 2, NPIX)
    smoothed = _conv_call(ph, ph, ph)
    weights = _gather_call(ind, smoothed.reshape(-1))
    return weights, mask


# speedup vs baseline: 1.0077x; 1.0077x over previous
import functools

import jax
import jax.numpy as jnp
from jax import lax
from jax.experimental import pallas as pl
from jax.experimental.pallas import tpu as pltpu
from jax.experimental.pallas import tpu_sc as plsc

NPIX = 2048
NCELL = NPIX * NPIX
N = 4194304
SENT = NCELL

HALF = NCELL // 2
PACKED = HALF // 2
NDUMP = 2048
HSIZE = PACKED + NDUMP
INIT_PER_TILE = HSIZE // 16
OUT_PER_TILE = PACKED // 16

B_H = 4096
SCAN_PER_TILE = N // 16

B_G = 4096
G_PER_TILE = N // 32


def _idx_body(u_ref, v_ref, ind_ref, mask_ref):
    u = u_ref[...]
    v = v_ref[...]
    fl = v < 0.0
    us = jnp.where(fl, -u, u)
    vs = jnp.where(fl, -v, v)
    p = jnp.floor((us + 1.0) * (NPIX / 2.0)).astype(jnp.int32)
    q = jnp.floor((vs + 1.0) * (NPIX / 2.0)).astype(jnp.int32)
    valid = (p >= 0) & (p < NPIX) & (q >= 0) & (q < NPIX)
    ind = jnp.where(valid, p * NPIX + jnp.where(valid, q, 0), SENT)
    ind_ref[...] = ind
    mask_ref[...] = valid


_IDX_BLOCK = 524288

_idx_call = pl.pallas_call(
    _idx_body,
    grid=(N // _IDX_BLOCK,),
    in_specs=[pl.BlockSpec((_IDX_BLOCK,), lambda i: (i,)),
              pl.BlockSpec((_IDX_BLOCK,), lambda i: (i,))],
    out_specs=[pl.BlockSpec((_IDX_BLOCK,), lambda i: (i,)),
               pl.BlockSpec((_IDX_BLOCK,), lambda i: (i,))],
    out_shape=[jax.ShapeDtypeStruct((N,), jnp.int32),
               jax.ShapeDtypeStruct((N,), jnp.bool_)],
)


_NCH = SCAN_PER_TILE // B_H


def _hist_body(ind_hbm, out_hbm, hist_sh,
               stage0, stage1, cidx0, cidx1, cval0, cval1,
               isem0, isem1, fsem0, fsem1, zsem):
    c = lax.axis_index("c")
    s = lax.axis_index("s")
    base = c * HALF
    iota = lax.iota(jnp.int32, 16)
    bufs = [(stage0, cidx0, cval0, isem0, fsem0),
            (stage1, cidx1, cval1, isem1, fsem1)]

    def zfill(j, _):
        cval0[pl.ds(j * 16, 16)] = jnp.zeros((16,), jnp.float32)
        return 0
    lax.fori_loop(0, B_H // 16, zfill, 0)

    zoff = s * INIT_PER_TILE
    for i in range(INIT_PER_TILE // B_H):
        pltpu.async_copy(cval0, hist_sh.at[pl.ds(zoff + i * B_H, B_H)],
                         zsem)
    pltpu.async_copy(cval0.at[pl.ds(0, INIT_PER_TILE % B_H)],
                     hist_sh.at[pl.ds(zoff + (INIT_PER_TILE // B_H) * B_H,
                                      INIT_PER_TILE % B_H)], zsem)
    for i in range(INIT_PER_TILE // B_H):
        pltpu.make_async_copy(cval0, hist_sh.at[pl.ds(0, B_H)],
                              zsem).wait()
    pltpu.make_async_copy(
        cval0.at[pl.ds(0, INIT_PER_TILE % B_H)],
        hist_sh.at[pl.ds(0, INIT_PER_TILE % B_H)], zsem).wait()
    plsc.subcore_barrier()

    dump_base = PACKED + s * 128

    def off(kk):
        return s * SCAN_PER_TILE + kk * B_H

    for b in range(2):
        stage, ci, cv, isem, fsem = bufs[b]
        pltpu.async_copy(ind_hbm.at[pl.ds(off(b), B_H)], stage, isem)

    def pair(k2, _):
        for b in range(2):
            stage, ci, cv, isem, fsem = bufs[b]
            kk = k2 * 2 + b
            pltpu.make_async_copy(
                ind_hbm.at[pl.ds(0, B_H)], stage, isem).wait()

            @pl.when(kk >= 2)
            def _credit():
                pltpu.make_async_copy(cv, hist_sh.at[ci], fsem).wait()

            def vec(j, _):
                iv = stage[pl.ds(j * 16, 16)]
                local = iv - base
                ok = (local >= 0) & (local < HALF)
                word = local & (PACKED - 1)
                dump = dump_base + ((j * 16 + iota) & 127)
                ci[pl.ds(j * 16, 16)] = jnp.where(ok, word, dump)
                cv[pl.ds(j * 16, 16)] = jnp.where(
                    ok, jnp.where(local >= PACKED, 4096.0, 1.0), 0.0)
                return 0
            lax.fori_loop(0, B_H // 16, vec, 0)

            pltpu.async_copy(cv, hist_sh.at[ci], fsem, add=True)

            @pl.when(kk + 2 < _NCH)
            def _prefetch():
                pltpu.async_copy(ind_hbm.at[pl.ds(off(kk + 2), B_H)],
                                 stage, isem)
        return 0
    lax.fori_loop(0, _NCH // 2, pair, 0)

    for b in range(2):
        stage, ci, cv, isem, fsem = bufs[b]
        pltpu.make_async_copy(cv, hist_sh.at[ci], fsem).wait()
    plsc.subcore_barrier()

    pltpu.sync_copy(hist_sh.at[pl.ds(s * OUT_PER_TILE, OUT_PER_TILE)],
                    out_hbm.at[pl.ds(c * PACKED + s * OUT_PER_TILE,
                                     OUT_PER_TILE)])


_hist_call = functools.partial(
    pl.kernel,
    out_type=jax.ShapeDtypeStruct((2 * PACKED,), jnp.float32),
    mesh=plsc.VectorSubcoreMesh(core_axis_name="c", subcore_axis_name="s"),
    scratch_types=[
        pltpu.VMEM_SHARED((HSIZE,), jnp.float32),
        pltpu.VMEM((B_H,), jnp.int32),
        pltpu.VMEM((B_H,), jnp.int32),
        pltpu.VMEM((B_H,), jnp.int32),
        pltpu.VMEM((B_H,), jnp.int32),
        pltpu.VMEM((B_H,), jnp.float32),
        pltpu.VMEM((B_H,), jnp.float32),
        pltpu.SemaphoreType.DMA,
        pltpu.SemaphoreType.DMA,
        pltpu.SemaphoreType.DMA,
        pltpu.SemaphoreType.DMA,
        pltpu.SemaphoreType.DMA,
    ],
)(_hist_body)


_BAND = 256
_NB = NPIX // _BAND


def _pk(k):
    return (k & 1) + 2 * (k >> 2)


def _use_b(k):
    return ((k >> 1) & 1) == 1


def _unpack(xp, use_b):
    b = jnp.floor(xp * (1.0 / 4096.0))
    a = xp - b * 4096.0
    return jnp.where(use_b, b, a)


def _conv_body(prev_ref, cur_ref, next_ref, o_ref):
    k = pl.program_id(0)
    x = _unpack(cur_ref[...], _use_b(k))
    top2 = _unpack(prev_ref[_BAND - 2:, :], _use_b(k - 1))
    top2 = jnp.where(k == 0, jnp.zeros_like(top2), top2)
    bot2 = _unpack(next_ref[:2, :], _use_b(k + 1))
    bot2 = jnp.where(k == _NB - 1, jnp.zeros_like(bot2), bot2)
    ext = jnp.concatenate([top2, x, bot2], axis=0)
    z1 = jnp.zeros((_BAND + 4, 1), jnp.float32)
    z2 = jnp.zeros((_BAND + 4, 2), jnp.float32)
    rs = ext
    rs = rs + jnp.concatenate([ext[:, 1:], z1], axis=1)
    rs = rs + jnp.concatenate([z1, ext[:, :-1]], axis=1)
    rs = rs + jnp.concatenate([ext[:, 2:], z2], axis=1)
    rs = rs + jnp.concatenate([z2, ext[:, :-2]], axis=1)
    cs = (rs[0:_BAND] + rs[1:_BAND + 1] + rs[2:_BAND + 2]
          + rs[3:_BAND + 3] + rs[4:_BAND + 4])
    o_ref[...] = cs * (1.0 / 25.0)


_conv_call = pl.pallas_call(
    _conv_body,
    grid=(_NB,),
    in_specs=[
        pl.BlockSpec((_BAND, NPIX), lambda k: (_pk(jnp.maximum(k - 1, 0)), 0)),
        pl.BlockSpec((_BAND, NPIX), lambda k: (_pk(k), 0)),
        pl.BlockSpec((_BAND, NPIX),
                     lambda k: (_pk(jnp.minimum(k + 1, _NB - 1)), 0)),
    ],
    out_specs=pl.BlockSpec((_BAND, NPIX), lambda k: (k, 0)),
    out_shape=jax.ShapeDtypeStruct((NPIX, NPIX), jnp.float32),
)


_NCG = G_PER_TILE // B_G


def _gather_body(ind_hbm, sm_hbm, w_hbm,
                 stage0, stage1, lidx0, lidx1, dens0, dens1, wbuf0, wbuf1,
                 isem0, isem1, gsem0, gsem1, osem0, osem1):
    c = lax.axis_index("c")
    s = lax.axis_index("s")
    wid = c * 16 + s
    bufs = [(stage0, lidx0, dens0, wbuf0, isem0, gsem0, osem0),
            (stage1, lidx1, dens1, wbuf1, isem1, gsem1, osem1)]

    def off(kk):
        return wid * G_PER_TILE + kk * B_G

    def compute_w(stage, dens, wbuf):
        def vec2(j, _):
            dv = dens[pl.ds(j * 16, 16)]
            iv = stage[pl.ds(j * 16, 16)]
            w = jnp.where(iv < SENT, 1.0 / jnp.maximum(dv, 1e-8), 0.0)
            wbuf[pl.ds(j * 16, 16)] = w
            return 0
        lax.fori_loop(0, B_G // 16, vec2, 0)

    pltpu.async_copy(ind_hbm.at[pl.ds(off(0), B_G)], stage0, isem0)

    def pair(k2, _):
        for b in range(2):
            stage, lidx, dens, wbuf, isem, gsem, osem = bufs[b]
            stage_q, lidx_q, dens_q, wbuf_q, isem_q, gsem_q, osem_q = \
                bufs[1 - b]
            kk = k2 * 2 + b
            pltpu.make_async_copy(
                ind_hbm.at[pl.ds(0, B_G)], stage, isem).wait()

            def vec1(j, _):
                iv = stage[pl.ds(j * 16, 16)]
                lidx[pl.ds(j * 16, 16)] = jnp.minimum(iv, NCELL - 1)
                return 0
            lax.fori_loop(0, B_G // 16, vec1, 0)
            pltpu.async_copy(sm_hbm.at[lidx], dens, gsem)

            @pl.when(kk >= 1)
            def _finish_prev():
                pltpu.make_async_copy(
                    sm_hbm.at[lidx_q], dens_q, gsem_q).wait()

                @pl.when(kk >= 3)
                def _drain_out():
                    pltpu.make_async_copy(
                        wbuf_q, w_hbm.at[pl.ds(0, B_G)], osem_q).wait()
                compute_w(stage_q, dens_q, wbuf_q)
                pltpu.async_copy(wbuf_q, w_hbm.at[pl.ds(off(kk - 1), B_G)],
                                 osem_q)

            @pl.when(kk + 1 < _NCG)
            def _prefetch():
                pltpu.async_copy(ind_hbm.at[pl.ds(off(kk + 1), B_G)],
                                 stage_q, isem_q)
        return 0
    lax.fori_loop(0, _NCG // 2, pair, 0)

    stage, lidx, dens, wbuf, isem, gsem, osem = bufs[(_NCG - 1) % 2]
    pltpu.make_async_copy(sm_hbm.at[lidx], dens, gsem).wait()
    pltpu.make_async_copy(wbuf, w_hbm.at[pl.ds(0, B_G)], osem).wait()
    compute_w(stage, dens, wbuf)
    pltpu.async_copy(wbuf, w_hbm.at[pl.ds(off(_NCG - 1), B_G)], osem)
    for b in range(2):
        stage, lidx, dens, wbuf, isem, gsem, osem = bufs[b]
        pltpu.make_async_copy(wbuf, w_hbm.at[pl.ds(0, B_G)], osem).wait()


_gather_call = functools.partial(
    pl.kernel,
    out_type=jax.ShapeDtypeStruct((N,), jnp.float32),
    mesh=plsc.VectorSubcoreMesh(core_axis_name="c", subcore_axis_name="s"),
    scratch_types=[
        pltpu.VMEM((B_G,), jnp.int32),
        pltpu.VMEM((B_G,), jnp.int32),
        pltpu.VMEM((B_G,), jnp.int32),
        pltpu.VMEM((B_G,), jnp.int32),
        pltpu.VMEM((B_G,), jnp.float32),
        pltpu.VMEM((B_G,), jnp.float32),
        pltpu.VMEM((B_G,), jnp.float32),
        pltpu.VMEM((B_G,), jnp.float32),
        pltpu.SemaphoreType.DMA,
        pltpu.SemaphoreType.DMA,
        pltpu.SemaphoreType.DMA,
        pltpu.SemaphoreType.DMA,
        pltpu.SemaphoreType.DMA,
        pltpu.SemaphoreType.DMA,
    ],
)(_gather_body)


def kernel(u, v):
    ind, mask = _idx_call(u, v)
    hist = _hist_call(ind)
    ph = hist.reshape(NPIX // 2, NPIX)
    smoothed = _conv_call(ph, ph, ph)
    weights = _gather_call(ind, smoothed.reshape(-1))
    return weights, mask
